# baseline (speedup 1.0000x reference)
import functools

import jax
import jax.numpy as jnp
from jax import lax
from jax.experimental import pallas as pl
from jax.experimental.pallas import tpu as pltpu
from jax.experimental.pallas import tpu_sc as plsc

_SCALE = 10.0
_K = 10
_TEMP = 0.5


def _align_body(feat_ref, txt_ref, aligned_ref, kl_ref):
    a = feat_ref[0]
    t = txt_ref[0]
    b = a.shape[0]
    an = jnp.sqrt(jnp.sum(a * a, axis=1, keepdims=True))
    a = a / jnp.maximum(an, 1e-12)
    tn = jnp.sqrt(jnp.sum(t * t, axis=1, keepdims=True))
    t = t / jnp.maximum(tn, 1e-12)
    si = lax.dot_general(a, a, (((1,), (1,)), ((), ())),
                         precision=lax.Precision.HIGHEST,
                         preferred_element_type=jnp.float32) * (1.0 / _TEMP)
    st = lax.dot_general(t, t, (((1,), (1,)), ((), ())),
                         precision=lax.Precision.HIGHEST,
                         preferred_element_type=jnp.float32) * (1.0 / _TEMP)

    def logsoftmax(s):
        m = jnp.max(s, axis=1, keepdims=True)
        e = jnp.exp(s - m)
        lse = jnp.log(jnp.sum(e, axis=1, keepdims=True)) + m
        return s - lse

    li = logsoftmax(si)
    lt = logsoftmax(st)
    pi = jnp.exp(li)
    pt = jnp.exp(lt)
    kl1 = jnp.sum(pt * (lt - li)) / b
    kl2 = jnp.sum(pi * (li - lt)) / b
    kl_ref[...] = jnp.broadcast_to(0.5 * (kl1 + kl2), (1, 1, 1))
    aligned_ref[0] = a + (t - a) * 0.1


def _knn_body(feat_ref, cen_ref, pos_ref, topk_ref, rl_ref):
    a = feat_ref[0]
    c = cen_ref[0]
    r, ddim = a.shape
    n = c.shape[0]
    h = n // 2
    posf = pos_ref[0, 0, :][:, None]

    rn = jnp.sum(a * a, axis=1, keepdims=True)
    cn = lax.dot_general(jnp.ones((1, ddim), jnp.float32), c * c,
                         (((1,), (1,)), ((), ())),
                         precision=lax.Precision.HIGHEST,
                         preferred_element_type=jnp.float32)
    ac = lax.dot_general(a, c, (((1,), (1,)), ((), ())),
                         precision=lax.Precision.DEFAULT,
                         preferred_element_type=jnp.float32)
    colf = lax.broadcasted_iota(jnp.int32, (r, n), 1).astype(jnp.float32)
    d = jnp.where(colf == posf, jnp.inf, rn + cn - 2.0 * ac)
    y = jnp.log(jnp.sum(jnp.exp(-_SCALE * d), axis=1))

    d_lo, d_hi = d[:, :h], d[:, h:]
    p_lo, p_hi = colf[:, :h], colf[:, h:]
    swap = d_hi < d_lo
    f1 = jnp.where(swap, d_hi, d_lo)
    f2 = jnp.where(swap, d_lo, d_hi)
    p1 = jnp.where(swap, p_hi, p_lo)
    p2 = jnp.where(swap, p_lo, p_hi)

    xsum = jnp.zeros((r, 1), jnp.float32)
    pv = jnp.zeros((r, 128), jnp.int32)
    lane = lax.broadcasted_iota(jnp.int32, (r, 128), 1)
    m = jnp.min(f1, axis=1, keepdims=True)
    for k in range(_K):
        key = jnp.min(jnp.where(f1 == m, p1, jnp.inf),
                      axis=1, keepdims=True)
        xsum = xsum + jnp.exp(-_SCALE * m)
        fidx = jnp.where(key > posf, key - 1.0, key)
        pv = jnp.where(lane == k, fidx.astype(jnp.int32), pv)
        cond = p1 == key
        f1 = jnp.where(cond, f2, f1)
        p1 = jnp.where(cond, p2, p1)
        f2 = jnp.where(cond, jnp.inf, f2)
        if k < _K - 1:
            m = jnp.min(f1, axis=1, keepdims=True)
    x = jnp.log(xsum[:, 0])
    rl_ref[0, 0, :] = y - x
    topk_ref[0] = pv


def _sc_gather(idx_flat, table, n_items, n_table):
    info = plsc.get_sparse_core_info()
    nc, ns, lanes = info.num_cores, info.num_subcores, info.num_lanes
    nw = nc * ns
    per_w = n_items // nw

    @functools.partial(
        pl.kernel,
        mesh=plsc.VectorSubcoreMesh(core_axis_name="c", subcore_axis_name="s"),
        out_type=jax.ShapeDtypeStruct((n_items,), jnp.int32),
        scratch_types=[
            pltpu.VMEM((per_w,), jnp.int32),
            pltpu.VMEM((n_table,), jnp.int32),
            pltpu.VMEM((per_w,), jnp.int32),
        ],
        compiler_params=pltpu.CompilerParams(needs_layout_passes=False),
    )
    def k(idx_hbm, table_hbm, out_hbm, idx_v, table_v, out_v):
        wid = lax.axis_index("s") * nc + lax.axis_index("c")
        base = wid * per_w
        pltpu.sync_copy(idx_hbm.at[pl.ds(base, per_w)], idx_v)
        pltpu.sync_copy(table_hbm, table_v)

        def body(i, _):
            iv = idx_v[pl.ds(i * lanes, lanes)]
            out_v[pl.ds(i * lanes, lanes)] = plsc.load_gather(table_v, [iv])
            return 0

        lax.fori_loop(0, per_w // lanes, body, 0)
        pltpu.sync_copy(out_v, out_hbm.at[pl.ds(base, per_w)])

    return k(idx_flat, table)


def kernel(feature, text_feature, centers, text_centers, position,
           pm_camid, pm_vid, camid):
    p, b, dd = feature.shape
    n = centers.shape[1]
    txt = jnp.transpose(text_feature, (1, 0, 2))

    aligned, klp = pl.pallas_call(
        _align_body,
        grid=(p,),
        in_specs=[
            pl.BlockSpec((1, b, dd), lambda i: (i, 0, 0)),
            pl.BlockSpec((1, b, dd), lambda i: (i, 0, 0)),
        ],
        out_specs=[
            pl.BlockSpec((1, b, dd), lambda i: (i, 0, 0)),
            pl.BlockSpec((1, 1, 1), lambda i: (i, 0, 0)),
        ],
        out_shape=[
            jax.ShapeDtypeStruct((p, b, dd), jnp.float32),
            jax.ShapeDtypeStruct((p, 1, 1), jnp.float32),
        ],
    )(feature, txt)

    r = 256 if b % 256 == 0 else b
    nb = b // r
    posr = position.astype(jnp.float32).reshape(nb, 1, r)

    topk_pad, rowloss = pl.pallas_call(
        _knn_body,
        grid=(p, nb),
        in_specs=[
            pl.BlockSpec((1, r, dd), lambda i, j: (i, j, 0)),
            pl.BlockSpec((1, n, dd), lambda i, j: (i, 0, 0)),
            pl.BlockSpec((1, 1, r), lambda i, j: (j, 0, 0)),
        ],
        out_specs=[
            pl.BlockSpec((1, r, 128), lambda i, j: (i, j, 0)),
            pl.BlockSpec((1, 1, r), lambda i, j: (i * nb + j, 0, 0)),
        ],
        out_shape=[
            jax.ShapeDtypeStruct((p, b, 128), jnp.int32),
            jax.ShapeDtypeStruct((p * nb, 1, r), jnp.float32),
        ],
    )(aligned, centers, posr)

    fidx = topk_pad[:, :, :_K].reshape(-1)
    posvid = _sc_gather(fidx, pm_vid, p * b * _K, n).reshape(p, b, _K)

    lp = jnp.sum(rowloss.reshape(p, b), axis=1) / b
    lp = jnp.where(jnp.isnan(lp), 0.0, lp)
    loss = jnp.sum(lp) / p + 0.5 * jnp.sum(klp)
    return loss, posvid
